# baseline (speedup 1.0000x reference)
import jax
import jax.numpy as jnp
from jax import lax
from jax.experimental import pallas as pl
from jax.experimental.pallas import tpu as pltpu
from jax.experimental.pallas import tpu_sc as plsc

EMBED = 128
HIST = 50
BATCH = 4096
NC, NS = 2, 16
NW = NC * NS
B_TOTAL = BATCH * HIST
B_PER_W = B_TOTAL // NW
CHUNK = 128
N_CHUNKS = B_PER_W // CHUNK
NBUF = 2
ROUNDS = N_CHUNKS // NBUF


def _body(idx_hbm, table_hbm, out_hbm, idx_v, rows_v, gsem, ssem):
    wid = lax.axis_index("s") * NC + lax.axis_index("c")
    out_base = wid * B_PER_W
    pltpu.sync_copy(idx_hbm.at[wid], idx_v)

    for b in range(NBUF):
        pltpu.async_copy(table_hbm.at[idx_v.at[b]], rows_v.at[b], gsem[b])
    for b in range(NBUF):
        pltpu.make_async_copy(table_hbm.at[idx_v.at[b]], rows_v.at[b], gsem[b]).wait()
        pltpu.async_copy(
            rows_v.at[b], out_hbm.at[pl.ds(out_base + b * CHUNK, CHUNK)], ssem[b]
        )

    @pl.loop(1, ROUNDS)
    def _round(r):
        j0 = r * NBUF
        for b in range(NBUF):
            j = j0 + b
            pltpu.make_async_copy(
                rows_v.at[b], out_hbm.at[pl.ds(out_base + (j - NBUF) * CHUNK, CHUNK)],
                ssem[b],
            ).wait()
            pltpu.async_copy(table_hbm.at[idx_v.at[j]], rows_v.at[b], gsem[b])
        for b in range(NBUF):
            j = j0 + b
            pltpu.make_async_copy(table_hbm.at[idx_v.at[j]], rows_v.at[b], gsem[b]).wait()
            pltpu.async_copy(
                rows_v.at[b], out_hbm.at[pl.ds(out_base + j * CHUNK, CHUNK)], ssem[b]
            )

    for b in range(NBUF):
        j = (ROUNDS - 1) * NBUF + b
        pltpu.make_async_copy(
            rows_v.at[b], out_hbm.at[pl.ds(out_base + j * CHUNK, CHUNK)], ssem[b]
        ).wait()


def kernel(input, table):
    idx = input.T.reshape(NW, N_CHUNKS, CHUNK).astype(jnp.int32)
    mesh = plsc.VectorSubcoreMesh(
        core_axis_name="c", subcore_axis_name="s", num_cores=NC, num_subcores=NS
    )
    flat = pl.kernel(
        _body,
        out_type=jax.ShapeDtypeStruct((B_TOTAL, EMBED), jnp.float32),
        mesh=mesh,
        scratch_types=[
            pltpu.VMEM((N_CHUNKS, CHUNK), jnp.int32),
            pltpu.VMEM((NBUF, CHUNK, EMBED), jnp.float32),
            [pltpu.SemaphoreType.DMA] * NBUF,
            [pltpu.SemaphoreType.DMA] * NBUF,
        ],
    )(idx, table)
    return flat.reshape(HIST, BATCH, EMBED).transpose(1, 0, 2)


# speedup vs baseline: 1.0730x; 1.0730x over previous
import jax
import jax.numpy as jnp
from jax import lax
from jax.experimental import pallas as pl
from jax.experimental.pallas import tpu as pltpu
from jax.experimental.pallas import tpu_sc as plsc

EMBED = 128
HIST = 50
BATCH = 4096
NC, NS = 2, 16
NW = NC * NS
B_TOTAL = BATCH * HIST
B_PER_W = B_TOTAL // NW
CHUNK = 64
N_CHUNKS = B_PER_W // CHUNK
NBUF = 10
ROUNDS = N_CHUNKS // NBUF


def _body(idx_hbm, table_hbm, out_hbm, idx_v, rows_v, gsem, ssem):
    wid = lax.axis_index("s") * NC + lax.axis_index("c")
    out_base = wid * B_PER_W
    pltpu.sync_copy(idx_hbm.at[wid], idx_v)

    for b in range(NBUF):
        pltpu.async_copy(table_hbm.at[idx_v.at[b]], rows_v.at[b], gsem[b])
    for b in range(NBUF):
        pltpu.make_async_copy(table_hbm.at[idx_v.at[b]], rows_v.at[b], gsem[b]).wait()
        pltpu.async_copy(
            rows_v.at[b], out_hbm.at[pl.ds(out_base + b * CHUNK, CHUNK)], ssem[b]
        )

    @pl.loop(1, ROUNDS)
    def _round(r):
        j0 = r * NBUF
        for b in range(NBUF):
            j = j0 + b
            pltpu.make_async_copy(
                rows_v.at[b], out_hbm.at[pl.ds(out_base + (j - NBUF) * CHUNK, CHUNK)],
                ssem[b],
            ).wait()
            pltpu.async_copy(table_hbm.at[idx_v.at[j]], rows_v.at[b], gsem[b])
        for b in range(NBUF):
            j = j0 + b
            pltpu.make_async_copy(table_hbm.at[idx_v.at[j]], rows_v.at[b], gsem[b]).wait()
            pltpu.async_copy(
                rows_v.at[b], out_hbm.at[pl.ds(out_base + j * CHUNK, CHUNK)], ssem[b]
            )

    for b in range(NBUF):
        j = (ROUNDS - 1) * NBUF + b
        pltpu.make_async_copy(
            rows_v.at[b], out_hbm.at[pl.ds(out_base + j * CHUNK, CHUNK)], ssem[b]
        ).wait()


def kernel(input, table):
    idx = input.T.reshape(NW, N_CHUNKS, CHUNK).astype(jnp.int32)
    mesh = plsc.VectorSubcoreMesh(
        core_axis_name="c", subcore_axis_name="s", num_cores=NC, num_subcores=NS
    )
    flat = pl.kernel(
        _body,
        out_type=jax.ShapeDtypeStruct((B_TOTAL, EMBED), jnp.float32),
        mesh=mesh,
        scratch_types=[
            pltpu.VMEM((N_CHUNKS, CHUNK), jnp.int32),
            pltpu.VMEM((NBUF, CHUNK, EMBED), jnp.float32),
            [pltpu.SemaphoreType.DMA] * NBUF,
            [pltpu.SemaphoreType.DMA] * NBUF,
        ],
    )(idx, table)
    return flat.reshape(HIST, BATCH, EMBED).transpose(1, 0, 2)


# speedup vs baseline: 1.0760x; 1.0028x over previous
import jax
import jax.numpy as jnp
from jax import lax
from jax.experimental import pallas as pl
from jax.experimental.pallas import tpu as pltpu
from jax.experimental.pallas import tpu_sc as plsc

EMBED = 128
HIST = 50
BATCH = 4096
NC, NS = 2, 16
NW = NC * NS
B_TOTAL = BATCH * HIST
B_PER_W = B_TOTAL // NW
CHUNK = 80
N_CHUNKS = B_PER_W // CHUNK
NBUF = 10
ROUNDS = N_CHUNKS // NBUF


def _body(idx_hbm, table_hbm, out_hbm, idx_v, rows_v, gsem, ssem):
    wid = lax.axis_index("s") * NC + lax.axis_index("c")
    out_base = wid * B_PER_W
    pltpu.sync_copy(idx_hbm.at[wid], idx_v)

    for b in range(NBUF):
        pltpu.async_copy(table_hbm.at[idx_v.at[b]], rows_v.at[b], gsem[b])
    for b in range(NBUF):
        pltpu.make_async_copy(table_hbm.at[idx_v.at[b]], rows_v.at[b], gsem[b]).wait()
        pltpu.async_copy(
            rows_v.at[b], out_hbm.at[pl.ds(out_base + b * CHUNK, CHUNK)], ssem[b]
        )

    @pl.loop(1, ROUNDS)
    def _round(r):
        j0 = r * NBUF
        for b in range(NBUF):
            j = j0 + b
            pltpu.make_async_copy(
                rows_v.at[b], out_hbm.at[pl.ds(out_base + (j - NBUF) * CHUNK, CHUNK)],
                ssem[b],
            ).wait()
            pltpu.async_copy(table_hbm.at[idx_v.at[j]], rows_v.at[b], gsem[b])
        for b in range(NBUF):
            j = j0 + b
            pltpu.make_async_copy(table_hbm.at[idx_v.at[j]], rows_v.at[b], gsem[b]).wait()
            pltpu.async_copy(
                rows_v.at[b], out_hbm.at[pl.ds(out_base + j * CHUNK, CHUNK)], ssem[b]
            )

    for b in range(NBUF):
        j = (ROUNDS - 1) * NBUF + b
        pltpu.make_async_copy(
            rows_v.at[b], out_hbm.at[pl.ds(out_base + j * CHUNK, CHUNK)], ssem[b]
        ).wait()


def kernel(input, table):
    idx = input.T.reshape(NW, N_CHUNKS, CHUNK).astype(jnp.int32)
    mesh = plsc.VectorSubcoreMesh(
        core_axis_name="c", subcore_axis_name="s", num_cores=NC, num_subcores=NS
    )
    flat = pl.kernel(
        _body,
        out_type=jax.ShapeDtypeStruct((B_TOTAL, EMBED), jnp.float32),
        mesh=mesh,
        scratch_types=[
            pltpu.VMEM((N_CHUNKS, CHUNK), jnp.int32),
            pltpu.VMEM((NBUF, CHUNK, EMBED), jnp.float32),
            [pltpu.SemaphoreType.DMA] * NBUF,
            [pltpu.SemaphoreType.DMA] * NBUF,
        ],
    )(idx, table)
    return flat.reshape(HIST, BATCH, EMBED).transpose(1, 0, 2)
